# Initial kernel scaffold; baseline (speedup 1.0000x reference)
import jax
import jax.numpy as jnp
from jax.experimental import pallas as pl


def kernel(x, edge_index, batch):
    raise NotImplementedError("write your pallas kernel here")


# speedup vs baseline: 8.7577x; 8.7577x over previous
import functools

import jax
import jax.numpy as jnp
from jax import lax
from jax.experimental import pallas as pl
from jax.experimental.pallas import tpu as pltpu
from jax.experimental.pallas import tpu_sc as plsc

D = 128
E = 160000
NW = 32
EPW = E // NW
C = 200
NCHUNK = EPW // C


def _make_kernel():
    mesh = plsc.VectorSubcoreMesh(core_axis_name="c", subcore_axis_name="s")

    @functools.partial(
        pl.kernel,
        mesh=mesh,
        out_type=(
            jax.ShapeDtypeStruct((E, D), jnp.float32),
            jax.ShapeDtypeStruct((E,), jnp.int32),
        ),
        scratch_types=[
            pltpu.VMEM((EPW,), jnp.int32),
            pltpu.VMEM((EPW,), jnp.int32),
            pltpu.VMEM((C, D), jnp.float32),
            pltpu.VMEM((C, D), jnp.float32),
            pltpu.VMEM((C,), jnp.int32),
            pltpu.SemaphoreType.DMA,
            pltpu.SemaphoreType.DMA,
            pltpu.SemaphoreType.DMA,
        ],
    )
    def tp_pool(x_hbm, src_hbm, dst_hbm, batch_hbm, out_hbm, eb_hbm,
                src_v, dst_v, a_v, b_v, eb_v, sem_a, sem_b, sem_e):
        wid = lax.axis_index("s") * 2 + lax.axis_index("c")
        base = wid * EPW
        pltpu.sync_copy(src_hbm.at[pl.ds(base, EPW)], src_v)
        pltpu.sync_copy(dst_hbm.at[pl.ds(base, EPW)], dst_v)

        def chunk_body(ci, carry):
            off = ci * C
            ca = pltpu.async_copy(x_hbm.at[src_v.at[pl.ds(off, C)]], a_v, sem_a)
            cb = pltpu.async_copy(x_hbm.at[dst_v.at[pl.ds(off, C)]], b_v, sem_b)
            ce = pltpu.async_copy(batch_hbm.at[src_v.at[pl.ds(off, C)]], eb_v,
                                  sem_e)
            ca.wait()
            cb.wait()

            def row_body(i, rcarry):
                for s in range(D // 16):
                    sl = pl.ds(s * 16, 16)
                    a_v[i, sl] = a_v[i, sl] + b_v[i, sl]
                return rcarry

            lax.fori_loop(0, C, row_body, 0)
            pltpu.sync_copy(a_v, out_hbm.at[pl.ds(base + off, C)])
            ce.wait()
            pltpu.sync_copy(eb_v, eb_hbm.at[pl.ds(base + off, C)])
            return carry

        lax.fori_loop(0, NCHUNK, chunk_body, 0)

    return tp_pool


_tp_pool = _make_kernel()


def kernel(x, edge_index, batch):
    ei = edge_index[:, ::2].astype(jnp.int32)
    src = ei[0]
    dst = ei[1]
    batch_i32 = batch.astype(jnp.int32)
    tp_features, edge_batch = _tp_pool(x, src, dst, batch_i32)
    return tp_features, edge_batch.astype(batch.dtype)


# speedup vs baseline: 10.1891x; 1.1635x over previous
import functools

import jax
import jax.numpy as jnp
from jax import lax
from jax.experimental import pallas as pl
from jax.experimental.pallas import tpu as pltpu
from jax.experimental.pallas import tpu_sc as plsc

D = 128
E = 160000
NW = 32
EPW = E // NW
C = 200
NCHUNK = EPW // C
NPAIR = NCHUNK // 2


def _make_kernel():
    mesh = plsc.VectorSubcoreMesh(core_axis_name="c", subcore_axis_name="s")

    @functools.partial(
        pl.kernel,
        mesh=mesh,
        out_type=(
            jax.ShapeDtypeStruct((E, D), jnp.float32),
            jax.ShapeDtypeStruct((E,), jnp.int32),
        ),
        scratch_types=[
            pltpu.VMEM((EPW,), jnp.int32),
            pltpu.VMEM((EPW,), jnp.int32),
            pltpu.VMEM((C, D), jnp.float32),
            pltpu.VMEM((C, D), jnp.float32),
            pltpu.VMEM((C,), jnp.int32),
            pltpu.VMEM((C, D), jnp.float32),
            pltpu.VMEM((C, D), jnp.float32),
            pltpu.VMEM((C,), jnp.int32),
            pltpu.SemaphoreType.DMA,
            pltpu.SemaphoreType.DMA,
            pltpu.SemaphoreType.DMA,
            pltpu.SemaphoreType.DMA,
        ],
    )
    def tp_pool(x_hbm, src_hbm, dst_hbm, batch_hbm, out_hbm, eb_hbm,
                src_v, dst_v, a0, b0, e0, a1, b1, e1,
                sg0, sg1, ss0, ss1):
        wid = lax.axis_index("s") * 2 + lax.axis_index("c")
        base = wid * EPW
        pltpu.sync_copy(src_hbm.at[pl.ds(base, EPW)], src_v)
        pltpu.sync_copy(dst_hbm.at[pl.ds(base, EPW)], dst_v)

        def gather(ci, a, b, e, sem):
            off = ci * C
            pltpu.async_copy(x_hbm.at[src_v.at[pl.ds(off, C)]], a, sem)
            pltpu.async_copy(x_hbm.at[dst_v.at[pl.ds(off, C)]], b, sem)
            pltpu.async_copy(batch_hbm.at[src_v.at[pl.ds(off, C)]], e, sem)

        def wait_gather(a, b, e, sem):
            pltpu.make_async_copy(x_hbm.at[src_v.at[pl.ds(0, C)]], a, sem).wait()
            pltpu.make_async_copy(x_hbm.at[dst_v.at[pl.ds(0, C)]], b, sem).wait()
            pltpu.make_async_copy(batch_hbm.at[src_v.at[pl.ds(0, C)]], e,
                                  sem).wait()

        def add(a, b):
            def row_body(i, rcarry):
                for s in range(D // 16):
                    sl = pl.ds(s * 16, 16)
                    a[i, sl] = a[i, sl] + b[i, sl]
                return rcarry

            lax.fori_loop(0, C, row_body, 0)

        def store(ci, a, e, sem):
            off = base + ci * C
            pltpu.async_copy(a, out_hbm.at[pl.ds(off, C)], sem)
            pltpu.async_copy(e, eb_hbm.at[pl.ds(off, C)], sem)

        def wait_store(a, e, sem):
            pltpu.make_async_copy(a, out_hbm.at[pl.ds(base, C)], sem).wait()
            pltpu.make_async_copy(e, eb_hbm.at[pl.ds(base, C)], sem).wait()

        gather(0, a0, b0, e0, sg0)
        gather(1, a1, b1, e1, sg1)

        def pair_body(j, carry):
            ci = j * 2
            wait_gather(a0, b0, e0, sg0)
            add(a0, b0)
            store(ci, a0, e0, ss0)
            wait_gather(a1, b1, e1, sg1)
            add(a1, b1)
            store(ci + 1, a1, e1, ss1)
            wait_store(a0, e0, ss0)
            gather(ci + 2, a0, b0, e0, sg0)
            wait_store(a1, e1, ss1)

            @pl.when(ci + 3 < NCHUNK)
            def _():
                gather(ci + 3, a1, b1, e1, sg1)

            return carry

        lax.fori_loop(0, NPAIR, pair_body, 0)

        wait_gather(a0, b0, e0, sg0)
        add(a0, b0)
        store(NCHUNK - 1, a0, e0, ss0)
        wait_store(a0, e0, ss0)

    return tp_pool


_tp_pool = _make_kernel()


def kernel(x, edge_index, batch):
    ei = edge_index[:, ::2].astype(jnp.int32)
    src = ei[0]
    dst = ei[1]
    batch_i32 = batch.astype(jnp.int32)
    tp_features, edge_batch = _tp_pool(x, src, dst, batch_i32)
    return tp_features, edge_batch.astype(batch.dtype)


# speedup vs baseline: 13.6097x; 1.3357x over previous
import functools

import jax
import jax.numpy as jnp
from jax import lax
from jax.experimental import pallas as pl
from jax.experimental.pallas import tpu as pltpu
from jax.experimental.pallas import tpu_sc as plsc

D = 128
EI = 320000
E = EI // 2
NW = 32
EPW = E // NW
C = 200
NCHUNK = EPW // C
NPAIR = NCHUNK // 2
NGRP = (EPW + 15) // 16
EPW_PAD = NGRP * 16
STAGE = 2 * EPW_PAD


def _make_kernel():
    mesh = plsc.VectorSubcoreMesh(core_axis_name="c", subcore_axis_name="s")

    @functools.partial(
        pl.kernel,
        mesh=mesh,
        compiler_params=pltpu.CompilerParams(needs_layout_passes=False),
        out_type=(
            jax.ShapeDtypeStruct((E, D), jnp.float32),
            jax.ShapeDtypeStruct((E,), jnp.int32),
        ),
        scratch_types=[
            pltpu.VMEM((STAGE,), jnp.int32),
            pltpu.VMEM((EPW_PAD,), jnp.int32),
            pltpu.VMEM((EPW_PAD,), jnp.int32),
            pltpu.VMEM((C, D), jnp.float32),
            pltpu.VMEM((C, D), jnp.float32),
            pltpu.VMEM((C,), jnp.int32),
            pltpu.VMEM((C, D), jnp.float32),
            pltpu.VMEM((C, D), jnp.float32),
            pltpu.VMEM((C,), jnp.int32),
            pltpu.SemaphoreType.DMA,
            pltpu.SemaphoreType.DMA,
            pltpu.SemaphoreType.DMA,
            pltpu.SemaphoreType.DMA,
        ],
    )
    def tp_pool(x_hbm, ei_hbm, batch_hbm, out_hbm, eb_hbm,
                stage_v, src_v, dst_v, a0, b0, e0, a1, b1, e1,
                sg0, sg1, ss0, ss1):
        wid = lax.axis_index("s") * 2 + lax.axis_index("c")
        base = wid * EPW

        evens = lax.iota(jnp.int32, 16) * 2

        def compact(row_off, out_idx):
            pltpu.sync_copy(ei_hbm.at[pl.ds(row_off + 2 * base, 2 * EPW)],
                            stage_v.at[pl.ds(0, 2 * EPW)])

            def grp(g, carry):
                v = plsc.load_gather(stage_v, [evens + g * 32])
                out_idx[pl.ds(g * 16, 16)] = v
                return carry

            lax.fori_loop(0, NGRP, grp, 0)

        compact(0, src_v)
        compact(EI, dst_v)

        def gather(ci, a, b, e, sem):
            off = ci * C
            pltpu.async_copy(x_hbm.at[src_v.at[pl.ds(off, C)]], a, sem)
            pltpu.async_copy(x_hbm.at[dst_v.at[pl.ds(off, C)]], b, sem)
            pltpu.async_copy(batch_hbm.at[src_v.at[pl.ds(off, C)]], e, sem)

        def wait_gather(a, b, e, sem):
            pltpu.make_async_copy(x_hbm.at[src_v.at[pl.ds(0, C)]], a, sem).wait()
            pltpu.make_async_copy(x_hbm.at[dst_v.at[pl.ds(0, C)]], b, sem).wait()
            pltpu.make_async_copy(batch_hbm.at[src_v.at[pl.ds(0, C)]], e,
                                  sem).wait()

        def add(a, b):
            def row_body(i, rcarry):
                for s in range(D // 16):
                    sl = pl.ds(s * 16, 16)
                    plsc.addupdate(a.at[i, sl], b[i, sl])
                return rcarry

            lax.fori_loop(0, C, row_body, 0)

        def store(ci, a, e, sem):
            off = base + ci * C
            pltpu.async_copy(a, out_hbm.at[pl.ds(off, C)], sem)
            pltpu.async_copy(e, eb_hbm.at[pl.ds(off, C)], sem)

        def wait_store(a, e, sem):
            pltpu.make_async_copy(a, out_hbm.at[pl.ds(base, C)], sem).wait()
            pltpu.make_async_copy(e, eb_hbm.at[pl.ds(base, C)], sem).wait()

        gather(0, a0, b0, e0, sg0)
        gather(1, a1, b1, e1, sg1)

        def pair_body(j, carry):
            ci = j * 2
            wait_gather(a0, b0, e0, sg0)
            add(a0, b0)
            store(ci, a0, e0, ss0)
            wait_gather(a1, b1, e1, sg1)
            add(a1, b1)
            store(ci + 1, a1, e1, ss1)
            wait_store(a0, e0, ss0)
            gather(ci + 2, a0, b0, e0, sg0)
            wait_store(a1, e1, ss1)

            @pl.when(ci + 3 < NCHUNK)
            def _():
                gather(ci + 3, a1, b1, e1, sg1)

            return carry

        lax.fori_loop(0, NPAIR, pair_body, 0)

        wait_gather(a0, b0, e0, sg0)
        add(a0, b0)
        store(NCHUNK - 1, a0, e0, ss0)
        wait_store(a0, e0, ss0)

    return tp_pool


_tp_pool = _make_kernel()


def kernel(x, edge_index, batch):
    ei_flat = edge_index.astype(jnp.int32).reshape(-1)
    batch_i32 = batch.astype(jnp.int32)
    tp_features, edge_batch = _tp_pool(x, ei_flat, batch_i32)
    return tp_features, edge_batch.astype(batch.dtype)


# speedup vs baseline: 13.9409x; 1.0243x over previous
import functools

import jax
import jax.numpy as jnp
from jax import lax
from jax.experimental import pallas as pl
from jax.experimental.pallas import tpu as pltpu
from jax.experimental.pallas import tpu_sc as plsc

D = 128
EI = 320000
E = EI // 2
NW = 32
EPW = E // NW
C = 200
NCHUNK = EPW // C
NPAIR = NCHUNK // 2
NGRP = (EPW + 15) // 16
EPW_PAD = NGRP * 16
STAGE = 2 * EPW_PAD


def _make_kernel():
    mesh = plsc.VectorSubcoreMesh(core_axis_name="c", subcore_axis_name="s")

    @functools.partial(
        pl.kernel,
        mesh=mesh,
        compiler_params=pltpu.CompilerParams(needs_layout_passes=False),
        out_type=(
            jax.ShapeDtypeStruct((E, D), jnp.float32),
            jax.ShapeDtypeStruct((E,), jnp.int32),
        ),
        scratch_types=[
            pltpu.VMEM((STAGE,), jnp.int32),
            pltpu.VMEM((EPW_PAD,), jnp.int32),
            pltpu.VMEM((EPW_PAD,), jnp.int32),
            pltpu.VMEM((C, D), jnp.float32),
            pltpu.VMEM((C, D), jnp.float32),
            pltpu.VMEM((C,), jnp.int32),
            pltpu.VMEM((C, D), jnp.float32),
            pltpu.VMEM((C, D), jnp.float32),
            pltpu.VMEM((C,), jnp.int32),
            pltpu.SemaphoreType.DMA,
            pltpu.SemaphoreType.DMA,
            pltpu.SemaphoreType.DMA,
            pltpu.SemaphoreType.DMA,
            pltpu.SemaphoreType.DMA,
            pltpu.SemaphoreType.DMA,
        ],
    )
    def tp_pool(x_hbm, ei_hbm, batch_hbm, out_hbm, eb_hbm,
                stage_v, src_v, dst_v, s0, d0, e0, s1, d1, e1,
                gs0, gs1, gd0, gd1, ss0, ss1):
        wid = lax.axis_index("s") * 2 + lax.axis_index("c")
        base = wid * EPW

        evens = lax.iota(jnp.int32, 16) * 2

        def compact(row_off, out_idx):
            pltpu.sync_copy(ei_hbm.at[pl.ds(row_off + 2 * base, 2 * EPW)],
                            stage_v.at[pl.ds(0, 2 * EPW)])

            def grp(g, carry):
                v = plsc.load_gather(stage_v, [evens + g * 32])
                out_idx[pl.ds(g * 16, 16)] = v
                return carry

            lax.fori_loop(0, NGRP, grp, 0)

        compact(0, src_v)
        compact(EI, dst_v)

        def gather_src(ci, s, sem):
            pltpu.async_copy(x_hbm.at[src_v.at[pl.ds(ci * C, C)]], s, sem)

        def gather_dst(ci, d, e, sem):
            off = ci * C
            pltpu.async_copy(x_hbm.at[dst_v.at[pl.ds(off, C)]], d, sem)
            pltpu.async_copy(batch_hbm.at[src_v.at[pl.ds(off, C)]], e, sem)

        def wait_gather_src(s, sem):
            pltpu.make_async_copy(x_hbm.at[src_v.at[pl.ds(0, C)]], s, sem).wait()

        def wait_gather_dst(d, e, sem):
            pltpu.make_async_copy(x_hbm.at[dst_v.at[pl.ds(0, C)]], d, sem).wait()
            pltpu.make_async_copy(batch_hbm.at[src_v.at[pl.ds(0, C)]], e,
                                  sem).wait()

        def add(s, d):
            def row_body(i, rcarry):
                for k in range(D // 16):
                    sl = pl.ds(k * 16, 16)
                    plsc.addupdate(d.at[i, sl], s[i, sl])
                return rcarry

            lax.fori_loop(0, C, row_body, 0)

        def store(ci, d, e, sem):
            off = base + ci * C
            pltpu.async_copy(d, out_hbm.at[pl.ds(off, C)], sem)
            pltpu.async_copy(e, eb_hbm.at[pl.ds(off, C)], sem)

        def wait_store(d, e, sem):
            pltpu.make_async_copy(d, out_hbm.at[pl.ds(base, C)], sem).wait()
            pltpu.make_async_copy(e, eb_hbm.at[pl.ds(base, C)], sem).wait()

        gather_src(0, s0, gs0)
        gather_dst(0, d0, e0, gd0)
        gather_src(1, s1, gs1)
        gather_dst(1, d1, e1, gd1)

        def pair_body(j, carry):
            ci = j * 2
            wait_gather_src(s0, gs0)
            wait_gather_dst(d0, e0, gd0)
            add(s0, d0)
            gather_src(ci + 2, s0, gs0)
            store(ci, d0, e0, ss0)
            wait_gather_src(s1, gs1)
            wait_gather_dst(d1, e1, gd1)
            add(s1, d1)

            @pl.when(ci + 3 < NCHUNK)
            def _():
                gather_src(ci + 3, s1, gs1)

            store(ci + 1, d1, e1, ss1)
            wait_store(d0, e0, ss0)
            gather_dst(ci + 2, d0, e0, gd0)
            wait_store(d1, e1, ss1)

            @pl.when(ci + 3 < NCHUNK)
            def _():
                gather_dst(ci + 3, d1, e1, gd1)

            return carry

        lax.fori_loop(0, NPAIR, pair_body, 0)

        wait_gather_src(s0, gs0)
        wait_gather_dst(d0, e0, gd0)
        add(s0, d0)
        store(NCHUNK - 1, d0, e0, ss0)
        wait_store(d0, e0, ss0)

    return tp_pool


_tp_pool = _make_kernel()


def kernel(x, edge_index, batch):
    ei_flat = edge_index.astype(jnp.int32).reshape(-1)
    batch_i32 = batch.astype(jnp.int32)
    tp_features, edge_batch = _tp_pool(x, ei_flat, batch_i32)
    return tp_features, edge_batch.astype(batch.dtype)
